# initial kernel scaffold (unmeasured)
import jax
import jax.numpy as jnp
from jax import lax
from jax.experimental import pallas as pl
from jax.experimental.pallas import tpu as pltpu

N_DEV = 4


def kernel(partial, gamma):
    M, D = partial.shape[-2], partial.shape[-1]
    m_out = M // N_DEV
    x = partial.reshape(M, D).astype(jnp.bfloat16)
    g2 = gamma.reshape(1, D)

    def body(x_hbm, g_ref, out_ref, xv, send_buf, recv_buf,
             load_sem, send_sems, recv_sems):
        my = lax.axis_index("i")
        right = (my + 1) % N_DEV
        left = (my - 1) % N_DEV

        barrier_sem = pltpu.get_barrier_semaphore()
        for nbr in (left, right):
            pl.semaphore_signal(
                barrier_sem, inc=1,
                device_id=(nbr,), device_id_type=pl.DeviceIdType.MESH,
            )
        pl.semaphore_wait(barrier_sem, 2)

        def load_chunk(c):
            cp = pltpu.make_async_copy(
                x_hbm.at[pl.ds(c * m_out, m_out), :], xv, load_sem)
            cp.start()
            cp.wait()

        for s in range(N_DEV - 1):
            c = (my - 1 - s) % N_DEV
            load_chunk(c)
            if s == 0:
                send_buf[...] = xv[...]
            else:
                send_buf[...] = xv[...] + recv_buf[s - 1]
            rdma = pltpu.make_async_remote_copy(
                src_ref=send_buf,
                dst_ref=recv_buf.at[s],
                send_sem=send_sems.at[s],
                recv_sem=recv_sems.at[s],
                device_id=(right,),
                device_id_type=pl.DeviceIdType.MESH,
            )
            rdma.start()
            rdma.wait()

        load_chunk(my)
        y = (xv[...] + recv_buf[N_DEV - 2]).astype(jnp.float32)
        rms = jnp.sqrt(jnp.mean(y * y, axis=-1, keepdims=True) + 1e-6)
        out_ref[...] = y / rms * g_ref[...]

    return pl.pallas_call(
        body,
        out_shape=jax.ShapeDtypeStruct((m_out, D), jnp.float32),
        in_specs=[
            pl.BlockSpec(memory_space=pltpu.ANY),
            pl.BlockSpec(memory_space=pltpu.VMEM),
        ],
        out_specs=pl.BlockSpec(memory_space=pltpu.VMEM),
        scratch_shapes=[
            pltpu.VMEM((m_out, D), jnp.bfloat16),
            pltpu.VMEM((m_out, D), jnp.bfloat16),
            pltpu.VMEM((N_DEV - 1, m_out, D), jnp.bfloat16),
            pltpu.SemaphoreType.DMA,
            pltpu.SemaphoreType.DMA((N_DEV - 1,)),
            pltpu.SemaphoreType.DMA((N_DEV - 1,)),
        ],
        compiler_params=pltpu.CompilerParams(collective_id=0),
    )(x, g2)


# baseline (device time: 349922 ns/iter reference)
import jax
import jax.numpy as jnp
from jax import lax
from jax.experimental import pallas as pl
from jax.experimental.pallas import tpu as pltpu

N_DEV = 4
BLK = 512


def kernel(partial, gamma):
    M, D = partial.shape[-2], partial.shape[-1]
    m_out = M // N_DEV
    x = partial.reshape(M, D).astype(jnp.bfloat16)
    g2 = gamma.reshape(1, D)

    def body(x_hbm, g_ref, out_ref, send_buf, recv_buf,
             load_sem, send_sems, recv_sems, credit_sem):
        my = lax.axis_index("i")
        right = (my + 1) % N_DEV
        left = (my - 1) % N_DEV

        barrier_sem = pltpu.get_barrier_semaphore()
        for nbr in (left, right):
            pl.semaphore_signal(
                barrier_sem, inc=1,
                device_id=(nbr,), device_id_type=pl.DeviceIdType.MESH,
            )
        pl.semaphore_wait(barrier_sem, 2)

        def load_chunk(c):
            cp = pltpu.make_async_copy(
                x_hbm.at[pl.ds(c * m_out, m_out), :], send_buf, load_sem)
            cp.start()
            cp.wait()

        for s in range(N_DEV - 1):
            load_chunk((my - 1 - s) % N_DEV)
            if s > 0:
                send_buf[...] = send_buf[...] + recv_buf[(s - 1) % 2]
                if s + 1 <= N_DEV - 2:
                    pl.semaphore_signal(
                        credit_sem, inc=1,
                        device_id=(left,), device_id_type=pl.DeviceIdType.MESH,
                    )
            if s >= 2:
                pl.semaphore_wait(credit_sem, 1)
            rdma = pltpu.make_async_remote_copy(
                src_ref=send_buf,
                dst_ref=recv_buf.at[s % 2],
                send_sem=send_sems.at[s],
                recv_sem=recv_sems.at[s],
                device_id=(right,),
                device_id_type=pl.DeviceIdType.MESH,
            )
            rdma.start()
            rdma.wait()

        load_chunk(my)
        last = recv_buf.at[(N_DEV - 2) % 2]
        for b in range(m_out // BLK):
            rows = pl.ds(b * BLK, BLK)
            y = (send_buf[rows, :] + last[rows, :]).astype(jnp.float32)
            rms = jnp.sqrt(jnp.mean(y * y, axis=-1, keepdims=True) + 1e-6)
            out_ref[rows, :] = y / rms * g_ref[...]

    return pl.pallas_call(
        body,
        out_shape=jax.ShapeDtypeStruct((m_out, D), jnp.float32),
        in_specs=[
            pl.BlockSpec(memory_space=pltpu.MemorySpace.HBM),
            pl.BlockSpec(memory_space=pltpu.MemorySpace.VMEM),
        ],
        out_specs=pl.BlockSpec(memory_space=pltpu.MemorySpace.VMEM),
        scratch_shapes=[
            pltpu.VMEM((m_out, D), jnp.bfloat16),
            pltpu.VMEM((2, m_out, D), jnp.bfloat16),
            pltpu.SemaphoreType.DMA,
            pltpu.SemaphoreType.DMA((N_DEV - 1,)),
            pltpu.SemaphoreType.DMA((N_DEV - 1,)),
            pltpu.SemaphoreType.REGULAR,
        ],
        compiler_params=pltpu.CompilerParams(
            collective_id=0,
            vmem_limit_bytes=64 * 1024 * 1024,
        ),
    )(x, g2)


# device time: 214460 ns/iter; 1.6316x vs baseline; 1.6316x over previous
import jax
import jax.numpy as jnp
from jax import lax
from jax.experimental import pallas as pl
from jax.experimental.pallas import tpu as pltpu

N_DEV = 4
BLK = 512


def kernel(partial, gamma):
    M, D = partial.shape[-2], partial.shape[-1]
    m_out = M // N_DEV
    Dh = D // 2
    x = partial.reshape(M, D).astype(jnp.bfloat16)
    g2 = gamma.reshape(1, D)

    def body(x_hbm, g_ref, out_ref, send_r, send_l, recv_r, recv_l,
             load_sems, send_sems_r, recv_sems_r, send_sems_l, recv_sems_l,
             credit_r, credit_l):
        my = lax.axis_index("i")
        right = (my + 1) % N_DEV
        left = (my - 1) % N_DEV

        barrier_sem = pltpu.get_barrier_semaphore()
        for nbr in (left, right):
            pl.semaphore_signal(
                barrier_sem, inc=1,
                device_id=(nbr,), device_id_type=pl.DeviceIdType.MESH,
            )
        pl.semaphore_wait(barrier_sem, 2)

        def load_halves(c_r, c_l):
            a = pltpu.make_async_copy(
                x_hbm.at[pl.ds(c_r * m_out, m_out), pl.ds(0, Dh)],
                send_r, load_sems.at[0])
            b = pltpu.make_async_copy(
                x_hbm.at[pl.ds(c_l * m_out, m_out), pl.ds(Dh, Dh)],
                send_l, load_sems.at[1])
            a.start()
            b.start()
            a.wait()
            b.wait()

        for s in range(N_DEV - 1):
            load_halves((my - 1 - s) % N_DEV, (my + 1 + s) % N_DEV)
            if s > 0:
                send_r[...] = send_r[...] + recv_r[(s - 1) % 2]
                send_l[...] = send_l[...] + recv_l[(s - 1) % 2]
                if s + 1 <= N_DEV - 2:
                    pl.semaphore_signal(
                        credit_r, inc=1,
                        device_id=(left,), device_id_type=pl.DeviceIdType.MESH,
                    )
                    pl.semaphore_signal(
                        credit_l, inc=1,
                        device_id=(right,), device_id_type=pl.DeviceIdType.MESH,
                    )
            if s >= 2:
                pl.semaphore_wait(credit_r, 1)
                pl.semaphore_wait(credit_l, 1)
            rr = pltpu.make_async_remote_copy(
                src_ref=send_r,
                dst_ref=recv_r.at[s % 2],
                send_sem=send_sems_r.at[s],
                recv_sem=recv_sems_r.at[s],
                device_id=(right,),
                device_id_type=pl.DeviceIdType.MESH,
            )
            rl = pltpu.make_async_remote_copy(
                src_ref=send_l,
                dst_ref=recv_l.at[s % 2],
                send_sem=send_sems_l.at[s],
                recv_sem=recv_sems_l.at[s],
                device_id=(left,),
                device_id_type=pl.DeviceIdType.MESH,
            )
            rr.start()
            rl.start()
            rr.wait()
            rl.wait()

        load_halves(my, my)
        last_r = recv_r.at[(N_DEV - 2) % 2]
        last_l = recv_l.at[(N_DEV - 2) % 2]
        for b in range(m_out // BLK):
            rows = pl.ds(b * BLK, BLK)
            yl = (send_r[rows, :] + last_r[rows, :]).astype(jnp.float32)
            yr = (send_l[rows, :] + last_l[rows, :]).astype(jnp.float32)
            ss = (jnp.sum(yl * yl, axis=-1, keepdims=True)
                  + jnp.sum(yr * yr, axis=-1, keepdims=True))
            rms = jnp.sqrt(ss / D + 1e-6)
            out_ref[rows, 0:Dh] = yl / rms * g_ref[:, 0:Dh]
            out_ref[rows, Dh:D] = yr / rms * g_ref[:, Dh:D]

    return pl.pallas_call(
        body,
        out_shape=jax.ShapeDtypeStruct((m_out, D), jnp.float32),
        in_specs=[
            pl.BlockSpec(memory_space=pltpu.MemorySpace.HBM),
            pl.BlockSpec(memory_space=pltpu.MemorySpace.VMEM),
        ],
        out_specs=pl.BlockSpec(memory_space=pltpu.MemorySpace.VMEM),
        scratch_shapes=[
            pltpu.VMEM((m_out, Dh), jnp.bfloat16),
            pltpu.VMEM((m_out, Dh), jnp.bfloat16),
            pltpu.VMEM((2, m_out, Dh), jnp.bfloat16),
            pltpu.VMEM((2, m_out, Dh), jnp.bfloat16),
            pltpu.SemaphoreType.DMA((2,)),
            pltpu.SemaphoreType.DMA((N_DEV - 1,)),
            pltpu.SemaphoreType.DMA((N_DEV - 1,)),
            pltpu.SemaphoreType.DMA((N_DEV - 1,)),
            pltpu.SemaphoreType.DMA((N_DEV - 1,)),
            pltpu.SemaphoreType.REGULAR,
            pltpu.SemaphoreType.REGULAR,
        ],
        compiler_params=pltpu.CompilerParams(
            collective_id=0,
            vmem_limit_bytes=64 * 1024 * 1024,
        ),
    )(x, g2)


# device time: 198030 ns/iter; 1.7670x vs baseline; 1.0830x over previous
import jax
import jax.numpy as jnp
from jax import lax
from jax.experimental import pallas as pl
from jax.experimental.pallas import tpu as pltpu

N_DEV = 4
K = 4


def kernel(partial, gamma):
    M, D = partial.shape[-2], partial.shape[-1]
    m_out = M // N_DEV
    Dh = D // 2
    R = m_out // K
    x = partial.reshape(M, D).astype(jnp.bfloat16)
    g2 = gamma.reshape(1, D)

    def body(x_hbm, g_ref, out_ref, send_r, send_l, xbuf_r, xbuf_l,
             recv_r, recv_l, load_sems, send_sems_r, send_sems_l,
             recv_sems_r, recv_sems_l, credit_r, credit_l):
        my = lax.axis_index("i")
        right = (my + 1) % N_DEV
        left = (my - 1) % N_DEV

        D0 = dict(send=send_r, xbuf=xbuf_r, recv=recv_r,
                  ssem=send_sems_r, rsem=recv_sems_r,
                  dst=right, src=left, credit=credit_r, col0=0,
                  lsem=load_sems.at[0, 0], psem=load_sems.at[0, 1],
                  chunk=lambda s: (my - 1 - s) % N_DEV, xload=None)
        D1 = dict(send=send_l, xbuf=xbuf_l, recv=recv_l,
                  ssem=send_sems_l, rsem=recv_sems_l,
                  dst=left, src=right, credit=credit_l, col0=Dh,
                  lsem=load_sems.at[1, 0], psem=load_sems.at[1, 1],
                  chunk=lambda s: (my + 1 + s) % N_DEV, xload=None)
        DD = [D0, D1]

        def load(c, d, dstref, sem):
            cp = pltpu.make_async_copy(
                x_hbm.at[pl.ds(c * m_out, m_out), pl.ds(d['col0'], Dh)],
                dstref, sem)
            cp.start()
            return cp

        def sub_rdma(d, s, k):
            return pltpu.make_async_remote_copy(
                src_ref=d['send'].at[pl.ds(k * R, R), :],
                dst_ref=d['recv'].at[s % 2, pl.ds(k * R, R), :],
                send_sem=d['ssem'].at[k],
                recv_sem=d['rsem'].at[s % 2, k],
                device_id=(d['dst'],),
                device_id_type=pl.DeviceIdType.MESH,
            )

        pres = []
        for d in DD:
            pres.append(load(d['chunk'](0), d, d['send'], d['psem']))
            pres.append(load(d['chunk'](1), d, d['xbuf'], d['lsem']))

        barrier_sem = pltpu.get_barrier_semaphore()
        for nbr in (left, right):
            pl.semaphore_signal(
                barrier_sem, inc=1,
                device_id=(nbr,), device_id_type=pl.DeviceIdType.MESH,
            )
        pl.semaphore_wait(barrier_sem, 2)
        for p in pres:
            p.wait()

        for s in range(N_DEV - 1):
            for k in range(K):
                for d in DD:
                    if s > 0:
                        if k == 0 and d['xload'] is not None:
                            d['xload'].wait()
                            d['xload'] = None
                        prev = sub_rdma(d, s - 1, k)
                        prev.wait_recv()
                        prev.wait_send()
                        rows = pl.ds(k * R, R)
                        d['send'][rows, :] = (
                            d['xbuf'][rows, :]
                            + d['recv'][(s - 1) % 2, rows, :])
                        if k == K - 1 and s == 1:
                            pl.semaphore_signal(
                                d['credit'], inc=1,
                                device_id=(d['src'],),
                                device_id_type=pl.DeviceIdType.MESH,
                            )
                    if s == 2 and k == 0:
                        pl.semaphore_wait(d['credit'], 1)
                    sub_rdma(d, s, k).start()
            for d in DD:
                if s == 1:
                    d['xload'] = load(d['chunk'](2), d, d['xbuf'], d['lsem'])
                elif s == 2:
                    d['xload'] = load(my, d, d['xbuf'], d['lsem'])

        last = (N_DEV - 2) % 2
        for k in range(K):
            for d in DD:
                if k == 0:
                    d['xload'].wait()
                sub_rdma(d, 2, k).wait_recv()
            rows = pl.ds(k * R, R)
            yl = (xbuf_r[rows, :] + recv_r[last, rows, :]).astype(jnp.float32)
            yr = (xbuf_l[rows, :] + recv_l[last, rows, :]).astype(jnp.float32)
            ss = (jnp.sum(yl * yl, axis=-1, keepdims=True)
                  + jnp.sum(yr * yr, axis=-1, keepdims=True))
            rms = jnp.sqrt(ss / D + 1e-6)
            out_ref[rows, 0:Dh] = yl / rms * g_ref[:, 0:Dh]
            out_ref[rows, Dh:D] = yr / rms * g_ref[:, Dh:D]

        for d in DD:
            for k in range(K):
                sub_rdma(d, 2, k).wait_send()

    return pl.pallas_call(
        body,
        out_shape=jax.ShapeDtypeStruct((m_out, D), jnp.float32),
        in_specs=[
            pl.BlockSpec(memory_space=pltpu.MemorySpace.HBM),
            pl.BlockSpec(memory_space=pltpu.MemorySpace.VMEM),
        ],
        out_specs=pl.BlockSpec(memory_space=pltpu.MemorySpace.VMEM),
        scratch_shapes=[
            pltpu.VMEM((m_out, Dh), jnp.bfloat16),
            pltpu.VMEM((m_out, Dh), jnp.bfloat16),
            pltpu.VMEM((m_out, Dh), jnp.bfloat16),
            pltpu.VMEM((m_out, Dh), jnp.bfloat16),
            pltpu.VMEM((2, m_out, Dh), jnp.bfloat16),
            pltpu.VMEM((2, m_out, Dh), jnp.bfloat16),
            pltpu.SemaphoreType.DMA((2, 2)),
            pltpu.SemaphoreType.DMA((K,)),
            pltpu.SemaphoreType.DMA((K,)),
            pltpu.SemaphoreType.DMA((2, K)),
            pltpu.SemaphoreType.DMA((2, K)),
            pltpu.SemaphoreType.REGULAR,
            pltpu.SemaphoreType.REGULAR,
        ],
        compiler_params=pltpu.CompilerParams(
            collective_id=0,
            vmem_limit_bytes=64 * 1024 * 1024,
        ),
    )(x, g2)


# device time: 164573 ns/iter; 2.1262x vs baseline; 1.2033x over previous
import jax
import jax.numpy as jnp
from jax import lax
from jax.experimental import pallas as pl
from jax.experimental.pallas import tpu as pltpu

N_DEV = 4
K = 4


def kernel(partial, gamma):
    M, D = partial.shape[-2], partial.shape[-1]
    m_out = M // N_DEV
    Dh = D // 2
    R = m_out // K
    g2 = gamma.reshape(1, D)

    def body(x_hbm, g_ref, out_ref, send_r, send_l, xbuf_r, xbuf_l,
             recv_r, recv_l, load_sems, send_sems_r, send_sems_l,
             recv_sems_r, recv_sems_l, credit_r, credit_l):
        my = lax.axis_index("i")
        right = (my + 1) % N_DEV
        left = (my - 1) % N_DEV

        barrier_sem = pltpu.get_barrier_semaphore()
        for nbr in (left, right):
            pl.semaphore_signal(
                barrier_sem, inc=1,
                device_id=(nbr,), device_id_type=pl.DeviceIdType.MESH,
            )

        D0 = dict(send=send_r, xbuf=xbuf_r, recv=recv_r,
                  ssem=send_sems_r, rsem=recv_sems_r,
                  dst=right, src=left, credit=credit_r, col0=0,
                  lsem=load_sems.at[0],
                  chunk=lambda s: (my - 1 - s) % N_DEV, xload=None)
        D1 = dict(send=send_l, xbuf=xbuf_l, recv=recv_l,
                  ssem=send_sems_l, rsem=recv_sems_l,
                  dst=left, src=right, credit=credit_l, col0=Dh,
                  lsem=load_sems.at[1],
                  chunk=lambda s: (my + 1 + s) % N_DEV, xload=None)
        DD = [D0, D1]

        def load(c, d):
            cp = pltpu.make_async_copy(
                x_hbm.at[0, pl.ds(c * m_out, m_out), pl.ds(d['col0'], Dh)],
                d['xbuf'], d['lsem'])
            cp.start()
            return cp

        def sub_rdma(d, s, k):
            return pltpu.make_async_remote_copy(
                src_ref=d['send'].at[pl.ds(k * R, R), :],
                dst_ref=d['recv'].at[s % 2, pl.ds(k * R, R), :],
                send_sem=d['ssem'].at[k],
                recv_sem=d['rsem'].at[s % 2, k],
                device_id=(d['dst'],),
                device_id_type=pl.DeviceIdType.MESH,
            )

        for d in DD:
            d['xload'] = load(d['chunk'](0), d)

        for s in range(N_DEV - 1):
            for k in range(K):
                if s == 0 and k == 0:
                    pl.semaphore_wait(barrier_sem, 2)
                for d in DD:
                    if k == 0 and d['xload'] is not None:
                        d['xload'].wait()
                        d['xload'] = None
                    rows = pl.ds(k * R, R)
                    if s == 0:
                        d['send'][rows, :] = d['xbuf'][rows, :].astype(
                            jnp.bfloat16)
                    else:
                        prev = sub_rdma(d, s - 1, k)
                        prev.wait_recv()
                        prev.wait_send()
                        d['send'][rows, :] = (
                            d['xbuf'][rows, :]
                            + d['recv'][(s - 1) % 2, rows, :].astype(
                                jnp.float32)
                        ).astype(jnp.bfloat16)
                        if k == K - 1 and s == 1:
                            pl.semaphore_signal(
                                d['credit'], inc=1,
                                device_id=(d['src'],),
                                device_id_type=pl.DeviceIdType.MESH,
                            )
                    if s == 2 and k == 0:
                        pl.semaphore_wait(d['credit'], 1)
                    sub_rdma(d, s, k).start()
            for d in DD:
                d['xload'] = load(my if s == 2 else d['chunk'](s + 1), d)

        last = (N_DEV - 2) % 2
        for k in range(K):
            for d in DD:
                if k == 0:
                    d['xload'].wait()
                sub_rdma(d, 2, k).wait_recv()
            rows = pl.ds(k * R, R)
            yl = xbuf_r[rows, :] + recv_r[last, rows, :].astype(jnp.float32)
            yr = xbuf_l[rows, :] + recv_l[last, rows, :].astype(jnp.float32)
            ss = (jnp.sum(yl * yl, axis=-1, keepdims=True)
                  + jnp.sum(yr * yr, axis=-1, keepdims=True))
            rms = jnp.sqrt(ss / D + 1e-6)
            out_ref[rows, 0:Dh] = yl / rms * g_ref[:, 0:Dh]
            out_ref[rows, Dh:D] = yr / rms * g_ref[:, Dh:D]

        for d in DD:
            for k in range(K):
                sub_rdma(d, 2, k).wait_send()

    return pl.pallas_call(
        body,
        out_shape=jax.ShapeDtypeStruct((m_out, D), jnp.float32),
        in_specs=[
            pl.BlockSpec(memory_space=pltpu.MemorySpace.HBM),
            pl.BlockSpec(memory_space=pltpu.MemorySpace.VMEM),
        ],
        out_specs=pl.BlockSpec(memory_space=pltpu.MemorySpace.VMEM),
        scratch_shapes=[
            pltpu.VMEM((m_out, Dh), jnp.bfloat16),
            pltpu.VMEM((m_out, Dh), jnp.bfloat16),
            pltpu.VMEM((m_out, Dh), jnp.float32),
            pltpu.VMEM((m_out, Dh), jnp.float32),
            pltpu.VMEM((2, m_out, Dh), jnp.bfloat16),
            pltpu.VMEM((2, m_out, Dh), jnp.bfloat16),
            pltpu.SemaphoreType.DMA((2,)),
            pltpu.SemaphoreType.DMA((K,)),
            pltpu.SemaphoreType.DMA((K,)),
            pltpu.SemaphoreType.DMA((2, K)),
            pltpu.SemaphoreType.DMA((2, K)),
            pltpu.SemaphoreType.REGULAR,
            pltpu.SemaphoreType.REGULAR,
        ],
        compiler_params=pltpu.CompilerParams(
            collective_id=0,
            vmem_limit_bytes=64 * 1024 * 1024,
        ),
    )(partial, g2)


# device time: 160941 ns/iter; 2.1742x vs baseline; 1.0226x over previous
import jax
import jax.numpy as jnp
from jax import lax
from jax.experimental import pallas as pl
from jax.experimental.pallas import tpu as pltpu

N_DEV = 4
K = 4


def kernel(partial, gamma):
    M, D = partial.shape[-2], partial.shape[-1]
    m_out = M // N_DEV
    Dh = D // 2
    R = m_out // K
    g2 = gamma.reshape(1, D)

    def body(x_hbm, g_ref, out_ref, send_r, send_l, xbuf_r, xbuf_l,
             recv_r, recv_l, stage, load_sems, send_sems_r, send_sems_l,
             recv_sems_r, recv_sems_l, out_sems, credit_r, credit_l):
        my = lax.axis_index("i")
        right = (my + 1) % N_DEV
        left = (my - 1) % N_DEV

        barrier_sem = pltpu.get_barrier_semaphore()
        for nbr in (left, right):
            pl.semaphore_signal(
                barrier_sem, inc=1,
                device_id=(nbr,), device_id_type=pl.DeviceIdType.MESH,
            )

        D0 = dict(send=send_r, xbuf=xbuf_r, recv=recv_r,
                  ssem=send_sems_r, rsem=recv_sems_r,
                  dst=right, src=left, credit=credit_r, col0=0,
                  lsem=load_sems.at[0],
                  chunk=lambda s: (my - 1 - s) % N_DEV, xload=None)
        D1 = dict(send=send_l, xbuf=xbuf_l, recv=recv_l,
                  ssem=send_sems_l, rsem=recv_sems_l,
                  dst=left, src=right, credit=credit_l, col0=Dh,
                  lsem=load_sems.at[1],
                  chunk=lambda s: (my + 1 + s) % N_DEV, xload=None)
        DD = [D0, D1]

        def load(c, d):
            cp = pltpu.make_async_copy(
                x_hbm.at[0, pl.ds(c * m_out, m_out), pl.ds(d['col0'], Dh)],
                d['xbuf'], d['lsem'])
            cp.start()
            return cp

        def sub_rdma(d, s, k):
            return pltpu.make_async_remote_copy(
                src_ref=d['send'].at[pl.ds(k * R, R), :],
                dst_ref=d['recv'].at[s % 2, pl.ds(k * R, R), :],
                send_sem=d['ssem'].at[k],
                recv_sem=d['rsem'].at[s % 2, k],
                device_id=(d['dst'],),
                device_id_type=pl.DeviceIdType.MESH,
            )

        for d in DD:
            d['xload'] = load(d['chunk'](0), d)

        for s in range(N_DEV - 1):
            for k in range(K):
                if s == 0 and k == 0:
                    pl.semaphore_wait(barrier_sem, 2)
                for d in DD:
                    if k == 0 and d['xload'] is not None:
                        d['xload'].wait()
                        d['xload'] = None
                    rows = pl.ds(k * R, R)
                    if s == 0:
                        d['send'][rows, :] = d['xbuf'][rows, :].astype(
                            jnp.bfloat16)
                    else:
                        prev = sub_rdma(d, s - 1, k)
                        prev.wait_recv()
                        prev.wait_send()
                        d['send'][rows, :] = (
                            d['xbuf'][rows, :]
                            + d['recv'][(s - 1) % 2, rows, :].astype(
                                jnp.float32)
                        ).astype(jnp.bfloat16)
                        if k == K - 1 and s == 1:
                            pl.semaphore_signal(
                                d['credit'], inc=1,
                                device_id=(d['src'],),
                                device_id_type=pl.DeviceIdType.MESH,
                            )
                    if s == 2 and k == 0:
                        pl.semaphore_wait(d['credit'], 1)
                    sub_rdma(d, s, k).start()
            for d in DD:
                d['xload'] = load(my if s == 2 else d['chunk'](s + 1), d)

        last = (N_DEV - 2) % 2
        out_cps = {}
        for k in range(K):
            for d in DD:
                if k == 0:
                    d['xload'].wait()
                sub_rdma(d, 2, k).wait_recv()
            slot = k % 2
            if k >= 2:
                out_cps.pop(k - 2).wait()
            rows = pl.ds(k * R, R)
            yl = xbuf_r[rows, :] + recv_r[last, rows, :].astype(jnp.float32)
            yr = xbuf_l[rows, :] + recv_l[last, rows, :].astype(jnp.float32)
            ss = (jnp.sum(yl * yl, axis=-1, keepdims=True)
                  + jnp.sum(yr * yr, axis=-1, keepdims=True))
            rms = jnp.sqrt(ss / D + 1e-6)
            stage[slot, :, 0:Dh] = yl / rms * g_ref[:, 0:Dh]
            stage[slot, :, Dh:D] = yr / rms * g_ref[:, Dh:D]
            cp = pltpu.make_async_copy(
                stage.at[slot], out_ref.at[rows, :], out_sems.at[slot])
            cp.start()
            out_cps[k] = cp
        for cp in out_cps.values():
            cp.wait()

        for d in DD:
            for k in range(K):
                sub_rdma(d, 2, k).wait_send()

    return pl.pallas_call(
        body,
        out_shape=jax.ShapeDtypeStruct((m_out, D), jnp.float32),
        in_specs=[
            pl.BlockSpec(memory_space=pltpu.MemorySpace.HBM),
            pl.BlockSpec(memory_space=pltpu.MemorySpace.VMEM),
        ],
        out_specs=pl.BlockSpec(memory_space=pltpu.MemorySpace.HBM),
        scratch_shapes=[
            pltpu.VMEM((m_out, Dh), jnp.bfloat16),
            pltpu.VMEM((m_out, Dh), jnp.bfloat16),
            pltpu.VMEM((m_out, Dh), jnp.float32),
            pltpu.VMEM((m_out, Dh), jnp.float32),
            pltpu.VMEM((2, m_out, Dh), jnp.bfloat16),
            pltpu.VMEM((2, m_out, Dh), jnp.bfloat16),
            pltpu.VMEM((2, R, D), jnp.float32),
            pltpu.SemaphoreType.DMA((2,)),
            pltpu.SemaphoreType.DMA((K,)),
            pltpu.SemaphoreType.DMA((K,)),
            pltpu.SemaphoreType.DMA((2, K)),
            pltpu.SemaphoreType.DMA((2, K)),
            pltpu.SemaphoreType.DMA((2,)),
            pltpu.SemaphoreType.REGULAR,
            pltpu.SemaphoreType.REGULAR,
        ],
        compiler_params=pltpu.CompilerParams(
            collective_id=0,
            vmem_limit_bytes=64 * 1024 * 1024,
        ),
    )(partial, g2)
